# baseline (speedup 1.0000x reference)
import functools

import jax
import jax.numpy as jnp
from jax import lax
from jax.experimental import pallas as pl
from jax.experimental.pallas import tpu as pltpu
from jax.experimental.pallas import tpu_sc as plsc

B, L, D, K = 4, 1024, 256, 8192
N = B * L
TOK_TILE = 256
KC = 2048
N_TILES = N // TOK_TILE
LOSS_SCALE = 1.25 / (N * D)

SC_CORES = 2
SC_SUBCORES = 16
NW = SC_CORES * SC_SUBCORES
ROWS_PER_W = N // NW


def _dist_argmax_body(x_ref, cb_ref, idx_ref, loss_ref):
    ones8 = jnp.ones((8, D), jnp.float32)
    cns = []
    for c in range(K // KC):
        cb_c = cb_ref[pl.ds(c * KC, KC), :]
        sq = lax.dot_general(
            ones8, cb_c * cb_c, (((1,), (1,)), ((), ())),
            preferred_element_type=jnp.float32,
            precision=lax.Precision.HIGHEST,
        )
        cns.append(sq[0:1, :])

    loss_acc = jnp.zeros((1, 1), jnp.float32)
    for t in range(N_TILES):
        x_t = x_ref[pl.ds(t * TOK_TILE, TOK_TILE), :]
        x2 = x_t * 2.0
        xnorm = jnp.sum(x_t * x_t, axis=1, keepdims=True)

        best = jnp.full((TOK_TILE, 1), -jnp.inf, jnp.float32)
        besti = jnp.zeros((TOK_TILE, 1), jnp.int32)
        for c in range(K // KC):
            cb_c = cb_ref[pl.ds(c * KC, KC), :]
            s = lax.dot_general(
                x2, cb_c, (((1,), (1,)), ((), ())),
                preferred_element_type=jnp.float32,
                precision=lax.Precision.DEFAULT,
            ) - cns[c]
            m = jnp.max(s, axis=1, keepdims=True)
            it = lax.broadcasted_iota(jnp.int32, (TOK_TILE, KC), 1)
            am = jnp.min(jnp.where(s == m, it, KC), axis=1, keepdims=True) + c * KC
            upd = m > best
            besti = jnp.where(upd, am, besti)
            best = jnp.where(upd, m, best)

        idx_ref[pl.ds(t * TOK_TILE, TOK_TILE), :] = besti
        loss_acc += jnp.sum(xnorm - best, axis=0, keepdims=True)

    loss_ref[...] = loss_acc * LOSS_SCALE


_dist_argmax = pl.pallas_call(
    _dist_argmax_body,
    out_shape=[
        jax.ShapeDtypeStruct((N, 1), jnp.int32),
        jax.ShapeDtypeStruct((1, 1), jnp.float32),
    ],
)


def _gather_body(cb_hbm, idx_hbm, out_hbm, idx_v, rows_v, sem):
    wid = lax.axis_index("s") * SC_CORES + lax.axis_index("c")
    base = wid * ROWS_PER_W
    pltpu.sync_copy(idx_hbm.at[pl.ds(base, ROWS_PER_W)], idx_v)
    pltpu.async_copy(cb_hbm.at[idx_v], rows_v, sem).wait()
    pltpu.sync_copy(rows_v, out_hbm.at[pl.ds(base, ROWS_PER_W)])


@functools.cache
def _make_gather():
    return pl.kernel(
        _gather_body,
        mesh=plsc.VectorSubcoreMesh(core_axis_name="c", subcore_axis_name="s"),
        out_type=jax.ShapeDtypeStruct((N, D), jnp.float32),
        scratch_types=[
            pltpu.VMEM((ROWS_PER_W,), jnp.int32),
            pltpu.VMEM((ROWS_PER_W, D), jnp.float32),
            pltpu.SemaphoreType.DMA,
        ],
    )


def kernel(x, codebook):
    x_flat = x.reshape(N, D)
    idx2d, loss11 = _dist_argmax(x_flat, codebook)
    idx_flat = idx2d.reshape(N)
    xq = _make_gather()(codebook, idx_flat)
    return (xq.reshape(B, L, D), idx_flat.reshape(B, L), loss11.reshape(()))


# speedup vs baseline: 1.1928x; 1.1928x over previous
import functools

import jax
import jax.numpy as jnp
from jax import lax
from jax.experimental import pallas as pl
from jax.experimental.pallas import tpu as pltpu
from jax.experimental.pallas import tpu_sc as plsc

B, L, D, K = 4, 1024, 256, 8192
N = B * L
TOK_TILE = 256
KC = 2048
N_TILES = N // TOK_TILE
LOSS_SCALE = 1.25 / (N * D)

SC_CORES = 2
SC_SUBCORES = 16
NW = SC_CORES * SC_SUBCORES
ROWS_PER_W = N // NW


def _dist_argmax_body(x_ref, cb_ref, idx_ref, loss_ref, cn_ref):
    i = pl.program_id(0)

    @pl.when(i == 0)
    def _():
        ones8 = jnp.ones((8, D), jnp.float32)
        for c in range(K // KC):
            cb_c = cb_ref[pl.ds(c * KC, KC), :]
            sq = lax.dot_general(
                ones8, cb_c * cb_c, (((1,), (1,)), ((), ())),
                preferred_element_type=jnp.float32,
                precision=lax.Precision.HIGHEST,
            )
            cn_ref[:, pl.ds(c * KC, KC)] = sq[0:1, :]

    x_t = x_ref[...]
    x2 = x_t * 2.0
    xnorm = jnp.sum(x_t * x_t, axis=1, keepdims=True)

    s = lax.dot_general(
        x2, cb_ref[...], (((1,), (1,)), ((), ())),
        preferred_element_type=jnp.float32,
        precision=lax.Precision.DEFAULT,
    ) - cn_ref[...]
    m = jnp.max(s, axis=1, keepdims=True)
    am = jnp.argmax(s, axis=1)[:, None]

    idx_ref[...] = am
    part = jnp.sum(xnorm - m, axis=0, keepdims=True)

    @pl.when(i == 0)
    def _():
        loss_ref[...] = jnp.zeros_like(loss_ref)

    loss_ref[...] += part

    @pl.when(i == N_TILES - 1)
    def _():
        loss_ref[...] *= LOSS_SCALE


_dist_argmax = pl.pallas_call(
    _dist_argmax_body,
    grid=(N_TILES,),
    in_specs=[
        pl.BlockSpec((TOK_TILE, D), lambda i: (i, 0)),
        pl.BlockSpec((K, D), lambda i: (0, 0)),
    ],
    out_specs=[
        pl.BlockSpec((TOK_TILE, 1), lambda i: (i, 0)),
        pl.BlockSpec((1, 1), lambda i: (0, 0)),
    ],
    out_shape=[
        jax.ShapeDtypeStruct((N, 1), jnp.int32),
        jax.ShapeDtypeStruct((1, 1), jnp.float32),
    ],
    scratch_shapes=[pltpu.VMEM((1, K), jnp.float32)],
)


def _gather_body(cb_hbm, idx_hbm, out_hbm, idx_v, rows_v, sem):
    wid = lax.axis_index("s") * SC_CORES + lax.axis_index("c")
    base = wid * ROWS_PER_W
    pltpu.sync_copy(idx_hbm.at[pl.ds(base, ROWS_PER_W)], idx_v)
    pltpu.async_copy(cb_hbm.at[idx_v], rows_v, sem).wait()
    pltpu.sync_copy(rows_v, out_hbm.at[pl.ds(base, ROWS_PER_W)])


@functools.cache
def _make_gather():
    return pl.kernel(
        _gather_body,
        mesh=plsc.VectorSubcoreMesh(core_axis_name="c", subcore_axis_name="s"),
        out_type=jax.ShapeDtypeStruct((N, D), jnp.float32),
        scratch_types=[
            pltpu.VMEM((ROWS_PER_W,), jnp.int32),
            pltpu.VMEM((ROWS_PER_W, D), jnp.float32),
            pltpu.SemaphoreType.DMA,
        ],
    )


def kernel(x, codebook):
    x_flat = x.reshape(N, D)
    idx2d, loss11 = _dist_argmax(x_flat, codebook)
    idx_flat = idx2d.reshape(N)
    xq = _make_gather()(codebook, idx_flat)
    return (xq.reshape(B, L, D), idx_flat.reshape(B, L), loss11.reshape(()))


# speedup vs baseline: 1.2221x; 1.0246x over previous
import functools

import jax
import jax.numpy as jnp
from jax import lax
from jax.experimental import pallas as pl
from jax.experimental.pallas import tpu as pltpu
from jax.experimental.pallas import tpu_sc as plsc

B, L, D, K = 4, 1024, 256, 8192
N = B * L
TOK_TILE = 512
KC = 2048
N_TILES = N // TOK_TILE
LOSS_SCALE = 1.25 / (N * D)

SC_CORES = 2
SC_SUBCORES = 16
NW = SC_CORES * SC_SUBCORES
ROWS_PER_W = N // NW


def _dist_argmax_body(x_ref, cb_ref, idx_ref, loss_ref, cn_ref):
    i = pl.program_id(0)

    @pl.when(i == 0)
    def _():
        ones8 = jnp.ones((8, D), jnp.float32)
        for c in range(K // KC):
            cb_c = cb_ref[pl.ds(c * KC, KC), :]
            sq = lax.dot_general(
                ones8, cb_c * cb_c, (((1,), (1,)), ((), ())),
                preferred_element_type=jnp.float32,
                precision=lax.Precision.HIGHEST,
            )
            cn_ref[:, pl.ds(c * KC, KC)] = sq[0:1, :]

    x_t = x_ref[...]
    x2 = x_t * 2.0
    xnorm = jnp.sum(x_t * x_t, axis=1, keepdims=True)

    s = lax.dot_general(
        x2, cb_ref[...], (((1,), (1,)), ((), ())),
        preferred_element_type=jnp.float32,
        precision=lax.Precision.DEFAULT,
    ) - cn_ref[...]
    m = jnp.max(s, axis=1, keepdims=True)
    am = jnp.argmax(s, axis=1)[:, None]

    idx_ref[...] = am
    part = jnp.sum(xnorm - m, axis=0, keepdims=True)

    @pl.when(i == 0)
    def _():
        loss_ref[...] = jnp.zeros_like(loss_ref)

    loss_ref[...] += part

    @pl.when(i == N_TILES - 1)
    def _():
        loss_ref[...] *= LOSS_SCALE


_dist_argmax = pl.pallas_call(
    _dist_argmax_body,
    grid=(N_TILES,),
    in_specs=[
        pl.BlockSpec((TOK_TILE, D), lambda i: (i, 0)),
        pl.BlockSpec((K, D), lambda i: (0, 0)),
    ],
    out_specs=[
        pl.BlockSpec((TOK_TILE, 1), lambda i: (i, 0)),
        pl.BlockSpec((1, 1), lambda i: (0, 0)),
    ],
    out_shape=[
        jax.ShapeDtypeStruct((N, 1), jnp.int32),
        jax.ShapeDtypeStruct((1, 1), jnp.float32),
    ],
    scratch_shapes=[pltpu.VMEM((1, K), jnp.float32)],
)


def _gather_body(cb_hbm, idx_hbm, out_hbm, idx_v, rows_v, sem):
    wid = lax.axis_index("s") * SC_CORES + lax.axis_index("c")
    base = wid * ROWS_PER_W
    pltpu.sync_copy(idx_hbm.at[pl.ds(base, ROWS_PER_W)], idx_v)
    pltpu.async_copy(cb_hbm.at[idx_v], rows_v, sem).wait()
    pltpu.sync_copy(rows_v, out_hbm.at[pl.ds(base, ROWS_PER_W)])


@functools.cache
def _make_gather():
    return pl.kernel(
        _gather_body,
        mesh=plsc.VectorSubcoreMesh(core_axis_name="c", subcore_axis_name="s"),
        out_type=jax.ShapeDtypeStruct((N, D), jnp.float32),
        scratch_types=[
            pltpu.VMEM((ROWS_PER_W,), jnp.int32),
            pltpu.VMEM((ROWS_PER_W, D), jnp.float32),
            pltpu.SemaphoreType.DMA,
        ],
    )


def kernel(x, codebook):
    x_flat = x.reshape(N, D)
    idx2d, loss11 = _dist_argmax(x_flat, codebook)
    idx_flat = idx2d.reshape(N)
    xq = _make_gather()(codebook, idx_flat)
    return (xq.reshape(B, L, D), idx_flat.reshape(B, L), loss11.reshape(()))


# speedup vs baseline: 1.2369x; 1.0121x over previous
import functools

import jax
import jax.numpy as jnp
from jax import lax
from jax.experimental import pallas as pl
from jax.experimental.pallas import tpu as pltpu
from jax.experimental.pallas import tpu_sc as plsc

B, L, D, K = 4, 1024, 256, 8192
N = B * L
TOK_TILE = 512
KC = 2048
N_TILES = N // TOK_TILE
LOSS_SCALE = 1.25 / (N * D)

SC_CORES = 2
SC_SUBCORES = 16
NW = SC_CORES * SC_SUBCORES
ROWS_PER_W = N // NW


def _dist_argmax_body(x_ref, cb_ref, idx_ref, loss_ref, cn_ref, cbbf_ref):
    i = pl.program_id(0)

    @pl.when(i == 0)
    def _():
        ones8 = jnp.ones((8, D), jnp.float32)
        for c in range(K // KC):
            cb_c = cb_ref[pl.ds(c * KC, KC), :]
            sq = lax.dot_general(
                ones8, cb_c * cb_c, (((1,), (1,)), ((), ())),
                preferred_element_type=jnp.float32,
                precision=lax.Precision.HIGHEST,
            )
            cn_ref[:, pl.ds(c * KC, KC)] = sq[0:1, :]
            cbbf_ref[pl.ds(c * KC, KC), :] = cb_c.astype(jnp.bfloat16)

    x_t = x_ref[...]
    x2 = (x_t * 2.0).astype(jnp.bfloat16)
    xnorm = jnp.sum(x_t * x_t, axis=1, keepdims=True)

    s = lax.dot_general(
        x2, cbbf_ref[...], (((1,), (1,)), ((), ())),
        preferred_element_type=jnp.float32,
        precision=lax.Precision.DEFAULT,
    ) - cn_ref[...]
    m = jnp.max(s, axis=1, keepdims=True)
    am = jnp.argmax(s, axis=1)[:, None]

    idx_ref[...] = am
    part = jnp.sum(xnorm - m, axis=0, keepdims=True)

    @pl.when(i == 0)
    def _():
        loss_ref[...] = jnp.zeros_like(loss_ref)

    loss_ref[...] += part

    @pl.when(i == N_TILES - 1)
    def _():
        loss_ref[...] *= LOSS_SCALE


_dist_argmax = pl.pallas_call(
    _dist_argmax_body,
    grid=(N_TILES,),
    in_specs=[
        pl.BlockSpec((TOK_TILE, D), lambda i: (i, 0)),
        pl.BlockSpec((K, D), lambda i: (0, 0)),
    ],
    out_specs=[
        pl.BlockSpec((TOK_TILE, 1), lambda i: (i, 0)),
        pl.BlockSpec((1, 1), lambda i: (0, 0)),
    ],
    out_shape=[
        jax.ShapeDtypeStruct((N, 1), jnp.int32),
        jax.ShapeDtypeStruct((1, 1), jnp.float32),
    ],
    scratch_shapes=[
        pltpu.VMEM((1, K), jnp.float32),
        pltpu.VMEM((K, D), jnp.bfloat16),
    ],
)


def _gather_body(cb_hbm, idx_hbm, out_hbm, idx_v, rows_v, sem):
    wid = lax.axis_index("s") * SC_CORES + lax.axis_index("c")
    base = wid * ROWS_PER_W
    pltpu.sync_copy(idx_hbm.at[pl.ds(base, ROWS_PER_W)], idx_v)
    pltpu.async_copy(cb_hbm.at[idx_v], rows_v, sem).wait()
    pltpu.sync_copy(rows_v, out_hbm.at[pl.ds(base, ROWS_PER_W)])


@functools.cache
def _make_gather():
    return pl.kernel(
        _gather_body,
        mesh=plsc.VectorSubcoreMesh(core_axis_name="c", subcore_axis_name="s"),
        out_type=jax.ShapeDtypeStruct((N, D), jnp.float32),
        scratch_types=[
            pltpu.VMEM((ROWS_PER_W,), jnp.int32),
            pltpu.VMEM((ROWS_PER_W, D), jnp.float32),
            pltpu.SemaphoreType.DMA,
        ],
    )


def kernel(x, codebook):
    x_flat = x.reshape(N, D)
    idx2d, loss11 = _dist_argmax(x_flat, codebook)
    idx_flat = idx2d.reshape(N)
    xq = _make_gather()(codebook, idx_flat)
    return (xq.reshape(B, L, D), idx_flat.reshape(B, L), loss11.reshape(()))


# speedup vs baseline: 1.2911x; 1.0438x over previous
import functools

import jax
import jax.numpy as jnp
from jax import lax
from jax.experimental import pallas as pl
from jax.experimental.pallas import tpu as pltpu
from jax.experimental.pallas import tpu_sc as plsc

B, L, D, K = 4, 1024, 256, 8192
N = B * L
TOK_TILE = 512
HALF = 256
KC = 2048
N_TILES = N // TOK_TILE
LOSS_SCALE = 1.25 / (N * D)

SC_CORES = 2
SC_SUBCORES = 16
NW = SC_CORES * SC_SUBCORES
ROWS_PER_W = N // NW


def _dist_argmax_body(x_ref, cb_ref, idx_ref, loss_ref, cn_ref, cbbf_ref):
    i = pl.program_id(0)

    @pl.when(i == 0)
    def _():
        ones8 = jnp.ones((8, D), jnp.float32)
        for c in range(K // KC):
            cb_c = cb_ref[pl.ds(c * KC, KC), :]
            sq = lax.dot_general(
                ones8, cb_c * cb_c, (((1,), (1,)), ((), ())),
                preferred_element_type=jnp.float32,
                precision=lax.Precision.HIGHEST,
            )
            cn_ref[:, pl.ds(c * KC, KC)] = sq[0:1, :]
            cbbf_ref[pl.ds(c * KC, KC), :] = cb_c.astype(jnp.bfloat16)

    part = jnp.zeros((1, 1), jnp.float32)
    for h in range(TOK_TILE // HALF):
        x_t = x_ref[pl.ds(h * HALF, HALF), :]
        x2 = (x_t * 2.0).astype(jnp.bfloat16)
        xnorm = jnp.sum(x_t * x_t, axis=1, keepdims=True)

        s = lax.dot_general(
            x2, cbbf_ref[...], (((1,), (1,)), ((), ())),
            preferred_element_type=jnp.float32,
            precision=lax.Precision.DEFAULT,
        ) - cn_ref[...]
        m = jnp.max(s, axis=1, keepdims=True)
        am = jnp.argmax(s, axis=1)[:, None]

        idx_ref[pl.ds(h * HALF, HALF), :] = am
        part += jnp.sum(xnorm - m, axis=0, keepdims=True)

    @pl.when(i == 0)
    def _():
        loss_ref[...] = jnp.zeros_like(loss_ref)

    loss_ref[...] += part

    @pl.when(i == N_TILES - 1)
    def _():
        loss_ref[...] *= LOSS_SCALE


_dist_argmax = pl.pallas_call(
    _dist_argmax_body,
    grid=(N_TILES,),
    in_specs=[
        pl.BlockSpec((TOK_TILE, D), lambda i: (i, 0)),
        pl.BlockSpec((K, D), lambda i: (0, 0)),
    ],
    out_specs=[
        pl.BlockSpec((TOK_TILE, 1), lambda i: (i, 0)),
        pl.BlockSpec((1, 1), lambda i: (0, 0)),
    ],
    out_shape=[
        jax.ShapeDtypeStruct((N, 1), jnp.int32),
        jax.ShapeDtypeStruct((1, 1), jnp.float32),
    ],
    scratch_shapes=[
        pltpu.VMEM((1, K), jnp.float32),
        pltpu.VMEM((K, D), jnp.bfloat16),
    ],
)


def _gather_body(cb_hbm, idx_hbm, out_hbm, idx_v, rows_v, sem):
    wid = lax.axis_index("s") * SC_CORES + lax.axis_index("c")
    base = wid * ROWS_PER_W
    pltpu.sync_copy(idx_hbm.at[pl.ds(base, ROWS_PER_W)], idx_v)
    pltpu.async_copy(cb_hbm.at[idx_v], rows_v, sem).wait()
    pltpu.sync_copy(rows_v, out_hbm.at[pl.ds(base, ROWS_PER_W)])


@functools.cache
def _make_gather():
    return pl.kernel(
        _gather_body,
        mesh=plsc.VectorSubcoreMesh(core_axis_name="c", subcore_axis_name="s"),
        out_type=jax.ShapeDtypeStruct((N, D), jnp.float32),
        scratch_types=[
            pltpu.VMEM((ROWS_PER_W,), jnp.int32),
            pltpu.VMEM((ROWS_PER_W, D), jnp.float32),
            pltpu.SemaphoreType.DMA,
        ],
    )


def kernel(x, codebook):
    x_flat = x.reshape(N, D)
    idx2d, loss11 = _dist_argmax(x_flat, codebook)
    idx_flat = idx2d.reshape(N)
    xq = _make_gather()(codebook, idx_flat)
    return (xq.reshape(B, L, D), idx_flat.reshape(B, L), loss11.reshape(()))


# speedup vs baseline: 1.3095x; 1.0143x over previous
import functools

import jax
import jax.numpy as jnp
from jax import lax
from jax.experimental import pallas as pl
from jax.experimental.pallas import tpu as pltpu
from jax.experimental.pallas import tpu_sc as plsc

B, L, D, K = 4, 1024, 256, 8192
N = B * L
TOK_TILE = 1024
HALF = 256
KC = 2048
N_TILES = N // TOK_TILE
LOSS_SCALE = 1.25 / (N * D)

SC_CORES = 2
SC_SUBCORES = 16
NW = SC_CORES * SC_SUBCORES
ROWS_PER_W = N // NW


def _dist_argmax_body(x_ref, cb_ref, idx_ref, loss_ref, cn_ref, cbbf_ref):
    i = pl.program_id(0)

    @pl.when(i == 0)
    def _():
        ones8 = jnp.ones((8, D), jnp.float32)
        for c in range(K // KC):
            cb_c = cb_ref[pl.ds(c * KC, KC), :]
            sq = lax.dot_general(
                ones8, cb_c * cb_c, (((1,), (1,)), ((), ())),
                preferred_element_type=jnp.float32,
                precision=lax.Precision.HIGHEST,
            )
            cn_ref[:, pl.ds(c * KC, KC)] = sq[0:1, :]
            cbbf_ref[pl.ds(c * KC, KC), :] = cb_c.astype(jnp.bfloat16)

    part = jnp.zeros((1, 1), jnp.float32)
    for h in range(TOK_TILE // HALF):
        x_t = x_ref[pl.ds(h * HALF, HALF), :]
        x2 = (x_t * 2.0).astype(jnp.bfloat16)
        xnorm = jnp.sum(x_t * x_t, axis=1, keepdims=True)

        s = lax.dot_general(
            x2, cbbf_ref[...], (((1,), (1,)), ((), ())),
            preferred_element_type=jnp.float32,
            precision=lax.Precision.DEFAULT,
        ) - cn_ref[...]
        m = jnp.max(s, axis=1, keepdims=True)
        am = jnp.argmax(s, axis=1)[:, None]

        idx_ref[pl.ds(h * HALF, HALF), :] = am
        part += jnp.sum(xnorm - m, axis=0, keepdims=True)

    @pl.when(i == 0)
    def _():
        loss_ref[...] = jnp.zeros_like(loss_ref)

    loss_ref[...] += part

    @pl.when(i == N_TILES - 1)
    def _():
        loss_ref[...] *= LOSS_SCALE


_dist_argmax = pl.pallas_call(
    _dist_argmax_body,
    grid=(N_TILES,),
    in_specs=[
        pl.BlockSpec((TOK_TILE, D), lambda i: (i, 0)),
        pl.BlockSpec((K, D), lambda i: (0, 0)),
    ],
    out_specs=[
        pl.BlockSpec((TOK_TILE, 1), lambda i: (i, 0)),
        pl.BlockSpec((1, 1), lambda i: (0, 0)),
    ],
    out_shape=[
        jax.ShapeDtypeStruct((N, 1), jnp.int32),
        jax.ShapeDtypeStruct((1, 1), jnp.float32),
    ],
    scratch_shapes=[
        pltpu.VMEM((1, K), jnp.float32),
        pltpu.VMEM((K, D), jnp.bfloat16),
    ],
)


def _gather_body(cb_hbm, idx_hbm, out_hbm, idx_v, rows_v, sem):
    wid = lax.axis_index("s") * SC_CORES + lax.axis_index("c")
    base = wid * ROWS_PER_W
    pltpu.sync_copy(idx_hbm.at[pl.ds(base, ROWS_PER_W)], idx_v)
    pltpu.async_copy(cb_hbm.at[idx_v], rows_v, sem).wait()
    pltpu.sync_copy(rows_v, out_hbm.at[pl.ds(base, ROWS_PER_W)])


@functools.cache
def _make_gather():
    return pl.kernel(
        _gather_body,
        mesh=plsc.VectorSubcoreMesh(core_axis_name="c", subcore_axis_name="s"),
        out_type=jax.ShapeDtypeStruct((N, D), jnp.float32),
        scratch_types=[
            pltpu.VMEM((ROWS_PER_W,), jnp.int32),
            pltpu.VMEM((ROWS_PER_W, D), jnp.float32),
            pltpu.SemaphoreType.DMA,
        ],
    )


def kernel(x, codebook):
    x_flat = x.reshape(N, D)
    idx2d, loss11 = _dist_argmax(x_flat, codebook)
    idx_flat = idx2d.reshape(N)
    xq = _make_gather()(codebook, idx_flat)
    return (xq.reshape(B, L, D), idx_flat.reshape(B, L), loss11.reshape(()))
